# baseline (speedup 1.0000x reference)
import functools

import jax
import jax.numpy as jnp
from jax import lax
from jax.experimental import pallas as pl
from jax.experimental.pallas import tpu as pltpu
from jax.experimental.pallas import tpu_sc as plsc

EMBED_DIM = 64
COMMITMENT_COST = 0.25
ROW_BLOCK = 256

_NUM_CORES = 2
_NUM_SUBCORES = 16
_NUM_WORKERS = _NUM_CORES * _NUM_SUBCORES
_GATHER_CHUNK = 96


CODE_BLOCK = 512


def _argmin_body(n_row_blocks, n_code_blocks,
                 z_ref, embt_ref, idx_ref, loss_ref, best_val, best_idx):
    i = pl.program_id(0)
    j = pl.program_id(1)
    z = z_ref[...]
    embt = embt_ref[...]
    e_sq = jnp.sum(embt * embt, axis=0)
    prod = lax.dot_general(
        z, embt, (((1,), (0,)), ((), ())),
        preferred_element_type=jnp.float32,
    )
    scores = e_sq[None, :] - 2.0 * prod
    blk_min = jnp.min(scores, axis=1)
    cols = lax.broadcasted_iota(jnp.int32, scores.shape, 1)
    blk_arg = jnp.min(
        jnp.where(scores == blk_min[:, None], cols, CODE_BLOCK), axis=1
    ) + j * CODE_BLOCK

    @pl.when(j == 0)
    def _():
        best_val[0, :] = blk_min
        best_idx[0, :] = blk_arg

    @pl.when(j > 0)
    def _():
        prev_val = best_val[0, :]
        prev_idx = best_idx[0, :]
        better = blk_min < prev_val
        best_val[0, :] = jnp.where(better, blk_min, prev_val)
        best_idx[0, :] = jnp.where(better, blk_arg, prev_idx)

    @pl.when(jnp.logical_and(i == 0, j == 0))
    def _():
        loss_ref[0, 0] = 0.0

    @pl.when(j == n_code_blocks - 1)
    def _():
        idx_ref[0, 0, :] = best_idx[0, :]
        z_sq = jnp.sum(z * z, axis=1)
        loss_ref[0, 0] += jnp.sum(z_sq + best_val[0, :])

    @pl.when(jnp.logical_and(i == n_row_blocks - 1, j == n_code_blocks - 1))
    def _():
        loss_ref[0, 0] *= COMMITMENT_COST / (n_row_blocks * ROW_BLOCK * EMBED_DIM)


def _tc_argmin(flat_z, embedding):
    n_tokens = flat_z.shape[0]
    n_codes = embedding.shape[0]
    n_row_blocks = n_tokens // ROW_BLOCK
    n_code_blocks = n_codes // CODE_BLOCK
    idx3, loss = pl.pallas_call(
        functools.partial(_argmin_body, n_row_blocks, n_code_blocks),
        grid=(n_row_blocks, n_code_blocks),
        in_specs=[
            pl.BlockSpec((ROW_BLOCK, EMBED_DIM), lambda i, j: (i, 0)),
            pl.BlockSpec((EMBED_DIM, CODE_BLOCK), lambda i, j: (0, j)),
        ],
        out_specs=[
            pl.BlockSpec((1, 1, ROW_BLOCK), lambda i, j: (i, 0, 0)),
            pl.BlockSpec(memory_space=pltpu.SMEM),
        ],
        out_shape=[
            jax.ShapeDtypeStruct((n_row_blocks, 1, ROW_BLOCK), jnp.int32),
            jax.ShapeDtypeStruct((1, 1), jnp.float32),
        ],
        scratch_shapes=[
            pltpu.VMEM((1, ROW_BLOCK), jnp.float32),
            pltpu.VMEM((1, ROW_BLOCK), jnp.int32),
        ],
    )(flat_z, embedding.T)
    return idx3.reshape(n_tokens), loss[0, 0]


def _sc_gather(indices, table_padded):
    n_tokens = indices.shape[0]
    width = table_padded.shape[1]
    per_worker = n_tokens // _NUM_WORKERS
    n_chunks = per_worker // _GATHER_CHUNK
    mesh = plsc.VectorSubcoreMesh(core_axis_name="c", subcore_axis_name="s")

    @functools.partial(
        pl.kernel,
        mesh=mesh,
        out_type=jax.ShapeDtypeStruct((n_tokens, width), jnp.float32),
        scratch_types=[
            pltpu.VMEM((_GATHER_CHUNK,), jnp.int32),
            pltpu.VMEM((_GATHER_CHUNK, width), jnp.float32),
            pltpu.SemaphoreType.DMA,
        ],
    )
    def gather(idx_hbm, table_hbm, out_hbm, idx_v, rows_v, sem):
        wid = lax.axis_index("s") * _NUM_CORES + lax.axis_index("c")
        base = wid * per_worker
        for j in range(n_chunks):
            off = base + j * _GATHER_CHUNK
            pltpu.sync_copy(idx_hbm.at[pl.ds(off, _GATHER_CHUNK)], idx_v)
            pltpu.async_copy(table_hbm.at[idx_v], rows_v, sem).wait()
            pltpu.sync_copy(rows_v, out_hbm.at[pl.ds(off, _GATHER_CHUNK)])

    return gather(indices, table_padded)


def kernel(z, embedding):
    flat_z = z.reshape(-1, EMBED_DIM)
    indices, loss = _tc_argmin(flat_z, embedding)
    table_padded = jnp.pad(embedding, ((0, 0), (0, 128 - EMBED_DIM)))
    z_q = _sc_gather(indices, table_padded)[:, :EMBED_DIM]
    return z_q.reshape(z.shape), loss, indices
